# baseline (speedup 1.0000x reference)
import functools

import jax
import jax.numpy as jnp
from jax import lax
from jax.experimental import pallas as pl
from jax.experimental.pallas import tpu as pltpu
from jax.experimental.pallas import tpu_sc as plsc

N_TILES = 32
LANES = 16
C1 = 32
NQ = 4
NR = N_TILES // NQ
CQ = C1 // NQ
NPAIR = CQ // 2


def _tc_pre(pos_t, W1, b1, W2, b2, W3, b3, k_nbr):
    def body(pos_ref, w1_ref, b1_ref, w2_ref, b2_ref, w3_ref, b3_ref,
             p_ref, c_ref, wc_ref, bias_ref):
        w = w1_ref[...]
        ps = pos_ref[...]
        npad = ps.shape[1]
        p32 = (w[3, :][:, None] * ps[0:1, :]
               + w[4, :][:, None] * ps[1:2, :]
               + w[5, :][:, None] * ps[2:3, :])
        c_ref[...] = (w[0, :][:, None] * ps[0:1, :]
                      + w[1, :][:, None] * ps[1:2, :]
                      + w[2, :][:, None] * ps[2:3, :]
                      + b1_ref[...][:, None]).reshape(NQ, CQ, npad)
        pr = p32.reshape(NQ, 2, NPAIR, npad)
        au = lax.bitcast_convert_type(pr[:, 0], jnp.int32) + 0x8000
        bu = lax.bitcast_convert_type(pr[:, 1], jnp.int32) + 0x8000
        pk = jnp.bitwise_or(lax.shift_right_logical(au, 16),
                            jnp.bitwise_and(bu, jnp.int32(-65536)))
        p_ref[...] = pk.reshape(NQ, NPAIR * npad)
        hi = jax.lax.Precision.HIGHEST
        wc_ref[...] = jnp.dot(w2_ref[...], w3_ref[...], precision=hi,
                              preferred_element_type=jnp.float32)
        bias_ref[...] = (float(k_nbr)
                         * jnp.dot(b2_ref[...][None, :], w3_ref[...],
                                   precision=hi,
                                   preferred_element_type=jnp.float32)
                         + b3_ref[...][None, :])

    npad = pos_t.shape[1]
    return pl.pallas_call(
        body,
        out_shape=(jax.ShapeDtypeStruct((NQ, NPAIR * npad), jnp.int32),
                   jax.ShapeDtypeStruct((NQ, CQ, npad), jnp.float32),
                   jax.ShapeDtypeStruct((C1, C1), jnp.float32),
                   jax.ShapeDtypeStruct((1, C1), jnp.float32)),
    )(pos_t, W1, b1, W2, b2, W3, b3)


def _sc_segment_sum(p2, c3, nbr3, k_nbr):
    npr = nbr3.shape[1] // k_nbr
    npad = NR * npr
    n_grp = npr // LANES
    mesh = plsc.VectorSubcoreMesh(core_axis_name="c", subcore_axis_name="s")

    @functools.partial(
        pl.kernel,
        out_type=jax.ShapeDtypeStruct((NQ, CQ, npad), jnp.float32),
        mesh=mesh,
        compiler_params=pltpu.CompilerParams(needs_layout_passes=False),
        scratch_types=[
            pltpu.VMEM((NPAIR * npad,), jnp.int32),
            pltpu.VMEM((CQ, npr), jnp.float32),
            pltpu.VMEM((npr * k_nbr,), jnp.int32),
            pltpu.VMEM((CQ, npr), jnp.float32),
            pltpu.SemaphoreType.DMA,
        ],
    )
    def sc_kernel(p_hbm, c_hbm, nbr_hbm, out_hbm, p_v, c_v, nbr_v, out_v,
                  sem):
        wid = lax.axis_index("s") * 2 + lax.axis_index("c")
        q = wid // NR
        r = wid - q * NR
        base = r * npr

        cps = [pltpu.async_copy(p_hbm.at[q], p_v, sem),
               pltpu.async_copy(c_hbm.at[q, :, pl.ds(base, npr)], c_v, sem),
               pltpu.async_copy(nbr_hbm.at[r], nbr_v, sem)]
        for cp in cps:
            cp.wait()

        lane = lax.iota(jnp.int32, LANES)

        def g_body(g, carry):
            gs = pl.multiple_of(g * LANES, LANES)
            nbs = lax.shift_left(gs + lane, 4)
            ccs = [c_v[c, pl.ds(gs, LANES)] for c in range(CQ)]
            accs = [jnp.zeros((LANES,), jnp.float32)] * CQ
            for k in range(k_nbr):
                kv = jnp.bitwise_and(lane + k, k_nbr - 1)
                idx = plsc.load_gather(nbr_v, [nbs + kv])
                for u in range(NPAIR):
                    pv = plsc.load_gather(p_v, [idx + (u * npad)])
                    lo = plsc.bitcast(lax.shift_left(pv, 16), jnp.float32)
                    hi = plsc.bitcast(pv, jnp.float32)
                    accs[u] = accs[u] + jnp.maximum(lo + ccs[u], 0.0)
                    accs[u + NPAIR] = (accs[u + NPAIR]
                                       + jnp.maximum(hi + ccs[u + NPAIR],
                                                     0.0))
            for c in range(CQ):
                out_v[c, pl.ds(gs, LANES)] = accs[c]
            return carry

        lax.fori_loop(0, n_grp, g_body, 0)
        pltpu.sync_copy(out_v, out_hbm.at[q, :, pl.ds(base, npr)])

    return sc_kernel(p2, c3, nbr3)


def _tc_project(s_cm, wc, bias):
    def body(s_ref, wc_ref, bias_ref, o_ref):
        o_ref[...] = lax.dot_general(
            s_ref[...], wc_ref[...], (((0,), (0,)), ((), ())),
            preferred_element_type=jnp.float32) + bias_ref[...]

    npad = s_cm.shape[1]
    return pl.pallas_call(
        body,
        out_shape=jax.ShapeDtypeStruct((npad, C1), jnp.float32),
    )(s_cm, wc, bias)


def kernel(inp_pos, out_pos, inp_features, neighbors_index,
           W1, b1, W2, b2, W3, b3):
    n = inp_pos.shape[0]
    k_nbr = neighbors_index.shape[1]
    npad = ((n + LANES * N_TILES - 1) // (LANES * N_TILES)) * (LANES * N_TILES)

    pos_t = jnp.zeros((3, npad), jnp.float32).at[:, :n].set(
        inp_pos.astype(jnp.float32).T)
    nbr_raw = jnp.zeros((npad, k_nbr), jnp.int32).at[:n].set(neighbors_index)
    nbr_m = jnp.where(nbr_raw == 0, npad - 1, nbr_raw - 1)

    p2, c3, wc, bias = _tc_pre(pos_t, W1.astype(jnp.float32),
                               b1.astype(jnp.float32), W2, b2, W3, b3,
                               k_nbr)
    s3 = _sc_segment_sum(p2, c3,
                         nbr_m.reshape(NR, (npad // NR) * k_nbr), k_nbr)
    out = _tc_project(s3.reshape(C1, npad), wc, bias)
    return out[:n].reshape(n, 1, 1, C1)


# speedup vs baseline: 1.1461x; 1.1461x over previous
import functools

import jax
import jax.numpy as jnp
from jax import lax
from jax.experimental import pallas as pl
from jax.experimental.pallas import tpu as pltpu
from jax.experimental.pallas import tpu_sc as plsc

N_TILES = 32
LANES = 16
C1 = 32
NQ = 4
NR = N_TILES // NQ
CQ = C1 // NQ
NPAIR = CQ // 2


def _tc_pre(pos_t, W1, b1, W2, b2, W3, b3, k_nbr):
    def body(pos_ref, w1_ref, b1_ref, w2_ref, b2_ref, w3_ref, b3_ref,
             p_ref, c_ref, wc_ref, bias_ref):
        w = w1_ref[...]
        ps = pos_ref[...]
        npad = ps.shape[1]
        p32 = (w[3, :][:, None] * ps[0:1, :]
               + w[4, :][:, None] * ps[1:2, :]
               + w[5, :][:, None] * ps[2:3, :])
        c_ref[...] = (w[0, :][:, None] * ps[0:1, :]
                      + w[1, :][:, None] * ps[1:2, :]
                      + w[2, :][:, None] * ps[2:3, :]
                      + b1_ref[...][:, None]).reshape(NQ, CQ, npad)
        pr = p32.reshape(NQ, 2, NPAIR, npad)
        au = lax.bitcast_convert_type(pr[:, 0], jnp.int32) + 0x8000
        bu = lax.bitcast_convert_type(pr[:, 1], jnp.int32) + 0x8000
        pk = jnp.bitwise_or(lax.shift_right_logical(au, 16),
                            jnp.bitwise_and(bu, jnp.int32(-65536)))
        p_ref[...] = pk.reshape(NQ, NPAIR * npad)
        hi = jax.lax.Precision.HIGHEST
        wc_ref[...] = jnp.dot(w2_ref[...], w3_ref[...], precision=hi,
                              preferred_element_type=jnp.float32)
        bias_ref[...] = (float(k_nbr)
                         * jnp.dot(b2_ref[...][None, :], w3_ref[...],
                                   precision=hi,
                                   preferred_element_type=jnp.float32)
                         + b3_ref[...][None, :])

    npad = pos_t.shape[1]
    return pl.pallas_call(
        body,
        out_shape=(jax.ShapeDtypeStruct((NQ, NPAIR * npad), jnp.int32),
                   jax.ShapeDtypeStruct((NQ, CQ, npad), jnp.float32),
                   jax.ShapeDtypeStruct((C1, C1), jnp.float32),
                   jax.ShapeDtypeStruct((1, C1), jnp.float32)),
    )(pos_t, W1, b1, W2, b2, W3, b3)


def _sc_segment_sum(p2, c3, nbr_t):
    k_nbr = nbr_t.shape[0]
    npad = nbr_t.shape[1]
    npr = npad // NR
    n_grp = npr // LANES
    mesh = plsc.VectorSubcoreMesh(core_axis_name="c", subcore_axis_name="s")

    @functools.partial(
        pl.kernel,
        out_type=jax.ShapeDtypeStruct((NQ, CQ, npad), jnp.float32),
        mesh=mesh,
        compiler_params=pltpu.CompilerParams(needs_layout_passes=False),
        scratch_types=[
            pltpu.VMEM((NPAIR * npad,), jnp.int32),
            pltpu.VMEM((CQ, npr), jnp.float32),
            pltpu.VMEM((k_nbr, npr), jnp.int32),
            pltpu.VMEM((CQ, npr), jnp.float32),
            pltpu.SemaphoreType.DMA,
        ],
    )
    def sc_kernel(p_hbm, c_hbm, nbr_hbm, out_hbm, p_v, c_v, nbr_v, out_v,
                  sem):
        wid = lax.axis_index("s") * 2 + lax.axis_index("c")
        q = wid // NR
        r = wid - q * NR
        base = r * npr

        cps = [pltpu.async_copy(p_hbm.at[q], p_v, sem),
               pltpu.async_copy(c_hbm.at[q, :, pl.ds(base, npr)], c_v, sem),
               pltpu.async_copy(nbr_hbm.at[:, pl.ds(base, npr)], nbr_v, sem)]
        for cp in cps:
            cp.wait()

        def g_body(g, carry):
            gs = pl.multiple_of(g * LANES, LANES)
            ccs = [c_v[c, pl.ds(gs, LANES)] for c in range(CQ)]
            accs = [jnp.zeros((LANES,), jnp.float32)] * CQ
            for k in range(k_nbr):
                idx = nbr_v[k, pl.ds(gs, LANES)]
                for u in range(NPAIR):
                    pv = plsc.load_gather(p_v, [idx + (u * npad)])
                    lo = plsc.bitcast(lax.shift_left(pv, 16), jnp.float32)
                    hi = plsc.bitcast(pv, jnp.float32)
                    accs[u] = accs[u] + jnp.maximum(lo + ccs[u], 0.0)
                    accs[u + NPAIR] = (accs[u + NPAIR]
                                       + jnp.maximum(hi + ccs[u + NPAIR],
                                                     0.0))
            for c in range(CQ):
                out_v[c, pl.ds(gs, LANES)] = accs[c]
            return carry

        lax.fori_loop(0, n_grp, g_body, 0)
        pltpu.sync_copy(out_v, out_hbm.at[q, :, pl.ds(base, npr)])

    return sc_kernel(p2, c3, nbr_t)


def _tc_project(s_cm, wc, bias):
    def body(s_ref, wc_ref, bias_ref, o_ref):
        o_ref[...] = lax.dot_general(
            s_ref[...], wc_ref[...], (((0,), (0,)), ((), ())),
            preferred_element_type=jnp.float32) + bias_ref[...]

    npad = s_cm.shape[1]
    return pl.pallas_call(
        body,
        out_shape=jax.ShapeDtypeStruct((npad, C1), jnp.float32),
    )(s_cm, wc, bias)


def kernel(inp_pos, out_pos, inp_features, neighbors_index,
           W1, b1, W2, b2, W3, b3):
    n = inp_pos.shape[0]
    k_nbr = neighbors_index.shape[1]
    npad = ((n + LANES * N_TILES - 1) // (LANES * N_TILES)) * (LANES * N_TILES)

    pos_t = jnp.zeros((3, npad), jnp.float32).at[:, :n].set(
        inp_pos.astype(jnp.float32).T)
    nbr_raw = jnp.zeros((k_nbr, npad), jnp.int32).at[:, :n].set(
        neighbors_index.T)
    nbr_t = jnp.where(nbr_raw == 0, npad - 1, nbr_raw - 1)

    p2, c3, wc, bias = _tc_pre(pos_t, W1.astype(jnp.float32),
                               b1.astype(jnp.float32), W2, b2, W3, b3,
                               k_nbr)
    s3 = _sc_segment_sum(p2, c3, nbr_t)
    out = _tc_project(s3.reshape(C1, npad), wc, bias)
    return out[:n].reshape(n, 1, 1, C1)
